# Initial kernel scaffold; baseline (speedup 1.0000x reference)
import jax
import jax.numpy as jnp
from jax.experimental import pallas as pl


def kernel(x, edge_index, batch, W1, b1, W2, b2, W3, b3, W4, b4, lin1_W, lin1_b, lin2_W, lin2_b):
    raise NotImplementedError("write your pallas kernel here")


# speedup vs baseline: 3.1146x; 3.1146x over previous
import jax
import jax.numpy as jnp
from jax import lax
from jax.experimental import pallas as pl
from jax.experimental.pallas import tpu as pltpu
from jax.experimental.pallas import tpu_sc as plsc

N = 10000
E = 160000
D = 256
H = 512
G = 64

NP = 10240
NC, NS = 2, 16
NW = NC * NS
EPW = E // NW
EB = 128
EPW_PAD = 5120
NB_EDGE = EPW_PAD // EB
RPT = NP // NS
CG = 4
CW = H // CG
RB = 1024
PB = 256
F32 = jnp.float32


def _sc_kernels():
    if "agg" not in _SC_CACHE:
        mesh = plsc.VectorSubcoreMesh(
            core_axis_name="c", subcore_axis_name="s",
            num_cores=NC, num_subcores=NS)
        _SC_CACHE["agg"] = pl.kernel(
            _agg_body,
            out_type=jax.ShapeDtypeStruct((NC, NP, CW), F32),
            mesh=mesh,
            scratch_types=[
                pltpu.VMEM_SHARED((NP, CW), F32),
                pltpu.VMEM((EB,), jnp.int32),
                pltpu.VMEM((EB,), jnp.int32),
                pltpu.VMEM((EB, CW), F32),
                pltpu.SemaphoreType.DMA,
            ],
        )
    return _SC_CACHE


_SC_CACHE = {}


def _agg_body(y_hbm, srcp_hbm, dstp_hbm, zeros_hbm, p_hbm,
              acc_sh, src_v, dst_v, rows_v, sem):
    c = lax.axis_index("c")
    s = lax.axis_index("s")
    w = c * NS + s
    r0 = s * RPT
    pltpu.sync_copy(zeros_hbm.at[pl.ds(r0, RPT), :],
                    acc_sh.at[pl.ds(r0, RPT), :])
    plsc.subcore_barrier()

    def step(k, carry):
        pltpu.sync_copy(srcp_hbm.at[w, pl.ds(k * EB, EB)], src_v)
        pltpu.sync_copy(dstp_hbm.at[w, pl.ds(k * EB, EB)], dst_v)
        pltpu.async_copy(y_hbm.at[src_v], rows_v, sem).wait()
        pltpu.sync_copy(rows_v, acc_sh.at[dst_v], add=True)
        return carry

    lax.fori_loop(0, NB_EDGE, step, 0)
    plsc.subcore_barrier()
    pltpu.sync_copy(acc_sh.at[pl.ds(r0, RPT), :],
                    p_hbm.at[c, pl.ds(r0, RPT), :])


def _row_valid(i, rows):
    rid = i * rows + lax.broadcasted_iota(jnp.int32, (rows, 1), 0)
    return rid < N


def _t1_body(x_ref, w_ref, cnt_ref, y_ref, dinv_ref):
    i = pl.program_id(0)
    dinv = lax.rsqrt(cnt_ref[0][:, 0:1] + cnt_ref[1][:, 0:1] + 1.0)
    y = jnp.dot(x_ref[...], w_ref[...], preferred_element_type=F32)
    yv = jnp.where(_row_valid(i, RB), y * dinv, 0.0)
    for g in range(CG):
        y_ref[g] = yv[:, g * CW:(g + 1) * CW]
    dinv_ref[...] = dinv


def _tmid_body(p0_ref, p1_ref, p2_ref, p3_ref, yprev_ref, dinv_ref, b_ref,
               w_ref, y_ref):
    i = pl.program_id(0)
    dinv = dinv_ref[...]
    prefs = (p0_ref, p1_ref, p2_ref, p3_ref)
    y = None
    for g in range(CG):
        s_g = prefs[g][0] + prefs[g][1] + yprev_ref[g]
        u_g = jax.nn.relu(s_g * dinv + b_ref[0:1, g * CW:(g + 1) * CW])
        part = jnp.dot(u_g, w_ref[g * CW:(g + 1) * CW, :],
                       preferred_element_type=F32)
        y = part if y is None else y + part
    yv = jnp.where(_row_valid(i, RB), y * dinv, 0.0)
    for g in range(CG):
        y_ref[g] = yv[:, g * CW:(g + 1) * CW]


def _pool_body(p0_ref, p1_ref, p2_ref, p3_ref, yprev_ref, dinv_ref, b_ref,
               batch_ref, l1w_ref, l1b_ref, l2w_ref, l2b_ref, out_ref,
               pacc_ref):
    i = pl.program_id(0)
    nblk = pl.num_programs(0)

    @pl.when(i == 0)
    def _init():
        pacc_ref[...] = jnp.full((G, H), -jnp.inf, F32)

    dinv = dinv_ref[...]
    prefs = (p0_ref, p1_ref, p2_ref, p3_ref)
    cols = []
    for g in range(CG):
        s_g = prefs[g][0] + prefs[g][1] + yprev_ref[g]
        cols.append(jax.nn.relu(s_g * dinv + b_ref[0:1, g * CW:(g + 1) * CW]))
    h = jnp.concatenate(cols, axis=1)
    hv = jnp.where(_row_valid(i, PB), h, -jnp.inf)

    bf = batch_ref[...]
    g_lo = bf[0, 0].astype(jnp.int32)
    g_hi = bf[PB - 1, 0].astype(jnp.int32)

    def upd(g, acc):
        m = bf == g.astype(F32)
        bm = jnp.max(jnp.where(m, hv, -jnp.inf), axis=0, keepdims=True)
        sel = lax.broadcasted_iota(jnp.int32, (G, 1), 0) == g
        return jnp.maximum(acc, jnp.where(sel, bm, -jnp.inf))

    pacc_ref[...] = lax.fori_loop(g_lo, g_hi + 1, upd, pacc_ref[...])

    @pl.when(i == nblk - 1)
    def _head():
        p = pacc_ref[...]
        u = jax.nn.relu(
            jnp.dot(p, l1w_ref[...], preferred_element_type=F32) + l1b_ref[...])
        out_ref[...] = (jnp.dot(u, l2w_ref[...], preferred_element_type=F32)
                        + l2b_ref[...])


def _full(shape):
    return pl.BlockSpec(shape, lambda i: tuple(0 for _ in shape))


def _t1_call(xp, W1, cnt):
    grid = NP // RB
    return pl.pallas_call(
        _t1_body,
        grid=(grid,),
        in_specs=[
            pl.BlockSpec((RB, D), lambda i: (i, 0)),
            _full((D, H)),
            pl.BlockSpec((NC, RB, CW), lambda i: (0, i, 0)),
        ],
        out_specs=[
            pl.BlockSpec((CG, RB, CW), lambda i: (0, i, 0)),
            pl.BlockSpec((RB, 1), lambda i: (i, 0)),
        ],
        out_shape=[
            jax.ShapeDtypeStruct((CG, NP, CW), F32),
            jax.ShapeDtypeStruct((NP, 1), F32),
        ],
    )(xp, W1, cnt)


def _tmid_call(parts, yprev, dinv, b, W):
    grid = NP // RB
    pspec = pl.BlockSpec((NC, RB, CW), lambda i: (0, i, 0))
    return pl.pallas_call(
        _tmid_body,
        grid=(grid,),
        in_specs=[pspec, pspec, pspec, pspec,
                  pl.BlockSpec((CG, RB, CW), lambda i: (0, i, 0)),
                  pl.BlockSpec((RB, 1), lambda i: (i, 0)),
                  _full((1, H)),
                  _full((H, H))],
        out_specs=pl.BlockSpec((CG, RB, CW), lambda i: (0, i, 0)),
        out_shape=jax.ShapeDtypeStruct((CG, NP, CW), F32),
    )(*parts, yprev, dinv, b, W)


def _pool_call(parts, yprev, dinv, b, batchf, l1w, l1b, l2w, l2b):
    grid = NP // PB
    pspec = pl.BlockSpec((NC, PB, CW), lambda i: (0, i, 0))
    return pl.pallas_call(
        _pool_body,
        grid=(grid,),
        in_specs=[pspec, pspec, pspec, pspec,
                  pl.BlockSpec((CG, PB, CW), lambda i: (0, i, 0)),
                  pl.BlockSpec((PB, 1), lambda i: (i, 0)),
                  _full((1, H)),
                  pl.BlockSpec((PB, 1), lambda i: (i, 0)),
                  _full((H, H)),
                  _full((1, H)),
                  _full((H, 1)),
                  _full((1, 1))],
        out_specs=_full((G, 1)),
        out_shape=jax.ShapeDtypeStruct((G, 1), F32),
        scratch_shapes=[pltpu.VMEM((G, H), F32)],
    )(*parts, yprev, dinv, b, batchf, l1w, l1b, l2w, l2b)


def kernel(x, edge_index, batch, W1, b1, W2, b2, W3, b3, W4, b4,
           lin1_W, lin1_b, lin2_W, lin2_b):
    i32 = jnp.int32
    xp = jnp.zeros((NP, D), F32).at[:N].set(x)
    src = edge_index[0].astype(i32)
    dst = edge_index[1].astype(i32)
    srcp = jnp.full((NW, EPW_PAD), N, i32).at[:, :EPW].set(src.reshape(NW, EPW))
    dstp = jnp.full((NW, EPW_PAD), N, i32).at[:, :EPW].set(dst.reshape(NW, EPW))
    batchf = jnp.full((NP, 1), float(G - 1), F32).at[:N, 0].set(
        batch.astype(F32))
    zeros_cg = jnp.zeros((NP, CW), F32)
    ones_cg = jnp.ones((NP, CW), F32)
    b1r, b2r, b3r, b4r = (b.reshape(1, H) for b in (b1, b2, b3, b4))
    l1b = lin1_b.reshape(1, H)
    l2b = lin2_b.reshape(1, 1)

    sck = _sc_kernels()
    cnt = sck["agg"](ones_cg, srcp, dstp, zeros_cg)
    y, dinv = _t1_call(xp, W1, cnt)
    for (b_l, W_l) in ((b1r, W2), (b2r, W3), (b3r, W4)):
        parts = [sck["agg"](y[g], srcp, dstp, zeros_cg) for g in range(CG)]
        y = _tmid_call(parts, y, dinv, b_l, W_l)
    parts = [sck["agg"](y[g], srcp, dstp, zeros_cg) for g in range(CG)]
    return _pool_call(parts, y, dinv, b4r, batchf, lin1_W, l1b, lin2_W, l2b)
